# baseline (speedup 1.0000x reference)
import jax
import jax.numpy as jnp
from jax.experimental import pallas as pl
from jax.experimental.pallas import tpu as pltpu

_CH = 256
_NBUF = 8


def _make_body(H, S, D):
    npc = S // _CH
    total = H * npc

    def body(a_hbm, m_hbm, b_ref, o_ref, abuf, mstage, mbf, a_sem, m_sem):
        def a_copy(t, slot):
            r = t // H
            h = jax.lax.rem(t, H)
            return pltpu.make_async_copy(
                a_hbm.at[h, pl.ds(r * _CH, _CH), :],
                abuf.at[slot],
                a_sem.at[slot],
            )

        def m_copy(r, slot):
            return pltpu.make_async_copy(
                m_hbm.at[pl.ds(r * _CH, _CH), :],
                mstage.at[slot],
                m_sem.at[slot],
            )

        for r in range(min(2, npc)):
            m_copy(r, r).start()
        for t in range(_NBUF):
            a_copy(t, t).start()

        def step(t, carry):
            slot = jax.lax.rem(t, _NBUF)
            r = t // H
            h = jax.lax.rem(t, H)

            @pl.when(h == 0)
            def _():
                ms = jax.lax.rem(r, 2)
                m_copy(r, ms).wait()
                mbf[...] = mstage[ms].astype(jnp.bfloat16)

                @pl.when(r + 2 < npc)
                def _():
                    m_copy(r + 2, ms).start()

            a_copy(t, slot).wait()
            a_blk = abuf[slot].astype(jnp.bfloat16) * mbf[...]
            o_ref[h, pl.ds(r * _CH, _CH), :] = jnp.dot(
                a_blk, b_ref[h], preferred_element_type=jnp.float32)

            @pl.when(t + _NBUF < total)
            def _():
                a_copy(t + _NBUF, slot).start()

            return carry

        jax.lax.fori_loop(0, total, step, 0)

    return body


def kernel(a, mask, b):
    B, H, S, _ = a.shape
    D = b.shape[-1]
    a3 = a.reshape(H, S, S)
    m2 = mask.reshape(S, S)
    b3 = b.reshape(H, S, D)

    out = pl.pallas_call(
        _make_body(H, S, D),
        in_specs=[
            pl.BlockSpec(memory_space=pltpu.MemorySpace.HBM),
            pl.BlockSpec(memory_space=pltpu.MemorySpace.HBM),
            pl.BlockSpec(memory_space=pltpu.MemorySpace.VMEM),
        ],
        out_specs=pl.BlockSpec(memory_space=pltpu.MemorySpace.VMEM),
        out_shape=jax.ShapeDtypeStruct((H, S, D), jnp.float32),
        scratch_shapes=[
            pltpu.VMEM((_NBUF, _CH, S), jnp.float32),
            pltpu.VMEM((2, _CH, S), jnp.int32),
            pltpu.VMEM((_CH, S), jnp.bfloat16),
            pltpu.SemaphoreType.DMA((_NBUF,)),
            pltpu.SemaphoreType.DMA((2,)),
        ],
    )(a3, m2, b3)
    return out.reshape(B, H, S, D)


# speedup vs baseline: 1.0139x; 1.0139x over previous
import jax
import jax.numpy as jnp
from jax.experimental import pallas as pl
from jax.experimental.pallas import tpu as pltpu

_CH = 256
_NBUF = 4


def _make_body(H, S, D):
    npc = S // _CH
    total = H * npc

    def body(a_hbm, m_hbm, b_ref, o_ref, abuf, mstage, mbf, a_sem, m_sem):
        def a_copy(t, slot):
            h = t // npc
            r = jax.lax.rem(t, npc)
            return pltpu.make_async_copy(
                a_hbm.at[h, pl.ds(r * _CH, _CH), :],
                abuf.at[slot],
                a_sem.at[slot],
            )

        def m_copy(r, slot):
            return pltpu.make_async_copy(
                m_hbm.at[pl.ds(r * _CH, _CH), :],
                mstage.at[slot],
                m_sem.at[slot],
            )

        for r in range(min(2, npc)):
            m_copy(r, r).start()
        for t in range(_NBUF):
            a_copy(t, t).start()

        def step(t, carry):
            slot = jax.lax.rem(t, _NBUF)
            h = t // npc
            r = jax.lax.rem(t, npc)

            @pl.when(t < npc)
            def _():
                ms = jax.lax.rem(r, 2)
                m_copy(r, ms).wait()
                mbf[pl.ds(r * _CH, _CH), :] = mstage[ms].astype(jnp.bfloat16)

                @pl.when(r + 2 < npc)
                def _():
                    m_copy(r + 2, ms).start()

            a_copy(t, slot).wait()
            a_blk = (abuf[slot].astype(jnp.bfloat16)
                     * mbf[pl.ds(r * _CH, _CH), :])
            o_ref[h, pl.ds(r * _CH, _CH), :] = jnp.dot(
                a_blk, b_ref[h], preferred_element_type=jnp.float32)

            @pl.when(t + _NBUF < total)
            def _():
                a_copy(t + _NBUF, slot).start()

            return carry

        jax.lax.fori_loop(0, total, step, 0)

    return body


def kernel(a, mask, b):
    B, H, S, _ = a.shape
    D = b.shape[-1]
    a3 = a.reshape(H, S, S)
    m2 = mask.reshape(S, S)
    b3 = b.reshape(H, S, D)

    out = pl.pallas_call(
        _make_body(H, S, D),
        in_specs=[
            pl.BlockSpec(memory_space=pltpu.MemorySpace.HBM),
            pl.BlockSpec(memory_space=pltpu.MemorySpace.HBM),
            pl.BlockSpec(memory_space=pltpu.MemorySpace.VMEM),
        ],
        out_specs=pl.BlockSpec(memory_space=pltpu.MemorySpace.VMEM),
        out_shape=jax.ShapeDtypeStruct((H, S, D), jnp.float32),
        scratch_shapes=[
            pltpu.VMEM((_NBUF, _CH, S), jnp.float32),
            pltpu.VMEM((2, _CH, S), jnp.int32),
            pltpu.VMEM((S, S), jnp.bfloat16),
            pltpu.SemaphoreType.DMA((_NBUF,)),
            pltpu.SemaphoreType.DMA((2,)),
        ],
    )(a3, m2, b3)
    return out.reshape(B, H, S, D)


# speedup vs baseline: 1.0148x; 1.0010x over previous
import jax
import jax.numpy as jnp
from jax.experimental import pallas as pl
from jax.experimental.pallas import tpu as pltpu

_CH = 256
_NBUF = 4


def _make_body(H, S, D):
    npc = S // _CH
    total = H * npc
    nblocks = total // _NBUF

    def body(a_hbm, m_hbm, b_ref, o_ref,
             ab0, ab1, ab2, ab3, mstage, mbf, a_sem, m_sem):
        abufs = [ab0, ab1, ab2, ab3]

        def a_copy(t, j):
            h = t // npc
            r = jax.lax.rem(t, npc)
            return pltpu.make_async_copy(
                a_hbm.at[h, pl.ds(r * _CH, _CH), :],
                abufs[j],
                a_sem.at[j],
            )

        def m_copy(r, slot):
            return pltpu.make_async_copy(
                m_hbm.at[pl.ds(r * _CH, _CH), :],
                mstage.at[slot],
                m_sem.at[slot],
            )

        for r in range(min(2, npc)):
            m_copy(r, r).start()
        for j in range(_NBUF):
            a_copy(j, j).start()

        def step(tb, carry):
            for j in range(_NBUF):
                t = tb * _NBUF + j
                h = t // npc
                r = jax.lax.rem(t, npc)

                @pl.when(t < npc)
                def _():
                    ms = jax.lax.rem(r, 2)
                    m_copy(r, ms).wait()
                    mbf[pl.ds(r * _CH, _CH), :] = mstage[ms].astype(jnp.bfloat16)

                    @pl.when(r + 2 < npc)
                    def _():
                        m_copy(r + 2, ms).start()

                a_copy(t, j).wait()
                a_blk = (abufs[j][...].astype(jnp.bfloat16)
                         * mbf[pl.ds(r * _CH, _CH), :])
                o_ref[h, pl.ds(r * _CH, _CH), :] = jnp.dot(
                    a_blk, b_ref[h], preferred_element_type=jnp.float32)

                @pl.when(t + _NBUF < total)
                def _():
                    a_copy(t + _NBUF, j).start()

            return carry

        jax.lax.fori_loop(0, nblocks, step, 0)

    return body


def kernel(a, mask, b):
    B, H, S, _ = a.shape
    D = b.shape[-1]
    a3 = a.reshape(H, S, S)
    m2 = mask.reshape(S, S)
    b3 = b.reshape(H, S, D)

    out = pl.pallas_call(
        _make_body(H, S, D),
        in_specs=[
            pl.BlockSpec(memory_space=pltpu.MemorySpace.HBM),
            pl.BlockSpec(memory_space=pltpu.MemorySpace.HBM),
            pl.BlockSpec(memory_space=pltpu.MemorySpace.VMEM),
        ],
        out_specs=pl.BlockSpec(memory_space=pltpu.MemorySpace.VMEM),
        out_shape=jax.ShapeDtypeStruct((H, S, D), jnp.float32),
        scratch_shapes=[
            pltpu.VMEM((_CH, S), jnp.float32),
            pltpu.VMEM((_CH, S), jnp.float32),
            pltpu.VMEM((_CH, S), jnp.float32),
            pltpu.VMEM((_CH, S), jnp.float32),
            pltpu.VMEM((2, _CH, S), jnp.int32),
            pltpu.VMEM((S, S), jnp.bfloat16),
            pltpu.SemaphoreType.DMA((_NBUF,)),
            pltpu.SemaphoreType.DMA((2,)),
        ],
    )(a3, m2, b3)
    return out.reshape(B, H, S, D)
